# baseline (speedup 1.0000x reference)
import functools

import jax
import jax.numpy as jnp
from jax import lax
from jax.experimental import pallas as pl
from jax.experimental.pallas import tpu as pltpu
from jax.experimental.pallas import tpu_sc as plsc

_N = 16384
_F = 1024
_NC = 2
_NS = 16
_NW = _NC * _NS
_RPW = _N // _NW
_CH = 32
_NCHUNK = _RPW // _CH
_NBUF = 3
_LANES = 16


def _sc_body(x_hbm, m_hbm, o_hbm, xb, mb, xsem, osem):
    wid = lax.axis_index("s") * _NC + lax.axis_index("c")
    base = wid * _RPW

    pltpu.sync_copy(m_hbm.at[pl.ds(base, _RPW)], mb)

    def in_copy(chunk, slot):
        return pltpu.make_async_copy(
            x_hbm.at[pl.ds(base + chunk * _CH, _CH), :], xb.at[slot],
            xsem.at[slot])

    def out_copy(chunk, slot):
        return pltpu.make_async_copy(
            xb.at[slot], o_hbm.at[pl.ds(base + chunk * _CH, _CH), :],
            osem.at[slot])

    def compute(ch, b):
        for half in range(_CH // _LANES):
            mvec = mb[pl.ds(ch * _CH + half * _LANES, _LANES)]
            for r in range(_LANES):
                mval = mvec[r]
                row = half * _LANES + r

                @plsc.parallel_loop(0, _F, step=_LANES * 8, unroll=2)
                def _(cb):
                    for k in range(8):
                        sl = pl.ds(cb + k * _LANES, _LANES)
                        xb[b, row, sl] = xb[b, row, sl] * mval

    def step(ch, b):
        in_copy(ch, b).wait()
        compute(ch, b)
        out_copy(ch, b).start()
        prev = ch - 1
        nxt = ch + _NBUF - 1

        @pl.when(jnp.logical_and(prev >= 0, nxt < _NCHUNK))
        def _():
            out_copy(prev, (b - 1) % _NBUF).wait()
            in_copy(nxt, (b - 1) % _NBUF).start()

    for b in range(_NBUF):
        in_copy(b, b).start()

    _NMAIN = (_NCHUNK // _NBUF) * _NBUF

    @pl.loop(0, _NMAIN, step=_NBUF)
    def _(g):
        for b in range(_NBUF):
            step(g + b, b)

    for ch in range(_NMAIN, _NCHUNK):
        step(ch, ch % _NBUF)

    for ch in range(_NCHUNK - _NBUF, _NCHUNK):
        out_copy(ch, ch % _NBUF).wait()


def kernel(input_tensor, mask_tensor):
    B, T, F = input_tensor.shape
    x = input_tensor.reshape(_N, _F)
    m = mask_tensor.reshape(_N)
    mesh = plsc.VectorSubcoreMesh(core_axis_name="c", subcore_axis_name="s")
    out = pl.kernel(
        _sc_body,
        out_type=jax.ShapeDtypeStruct((_N, _F), jnp.float32),
        mesh=mesh,
        scratch_types=[
            pltpu.VMEM((_NBUF, _CH, _F), jnp.float32),
            pltpu.VMEM((_RPW,), jnp.float32),
            pltpu.SemaphoreType.DMA((_NBUF,)),
            pltpu.SemaphoreType.DMA((_NBUF,)),
        ],
    )(x, m)
    return out.reshape(B, T, F)


# speedup vs baseline: 1.1076x; 1.1076x over previous
import functools

import jax
import jax.numpy as jnp
from jax import lax
from jax.experimental import pallas as pl
from jax.experimental.pallas import tpu as pltpu
from jax.experimental.pallas import tpu_sc as plsc

_N = 16384
_F = 1024
_NC = 2
_NS = 16
_NW = _NC * _NS
_RPW = _N // _NW
_CH = 32
_NCHUNK = _RPW // _CH
_NBUF = 3
_LANES = 16


def _sc_body(x_hbm, m_hbm, o_hbm, xb, mb, xsem, osem):
    wid = lax.axis_index("s") * _NC + lax.axis_index("c")
    base = wid * _RPW

    pltpu.sync_copy(m_hbm.at[pl.ds(base, _RPW)], mb)

    def in_copy(chunk, slot):
        return pltpu.make_async_copy(
            x_hbm.at[pl.ds(base + chunk * _CH, _CH), :], xb.at[slot],
            xsem.at[slot])

    def out_copy(chunk, slot):
        return pltpu.make_async_copy(
            xb.at[slot], o_hbm.at[pl.ds(base + chunk * _CH, _CH), :],
            osem.at[slot])

    def compute(ch, b):
        for half in range(_CH // _LANES):
            mvec = mb[pl.ds(ch * _CH + half * _LANES, _LANES)]
            for r in range(_LANES):
                mval = mvec[r]
                row = half * _LANES + r

                @pl.loop(0, _F, step=_LANES * 8)
                def _(cb):
                    for k in range(8):
                        sl = pl.ds(cb + k * _LANES, _LANES)
                        xb[b, row, sl] = xb[b, row, sl] * mval

    def step(ch, b):
        in_copy(ch, b).wait()
        compute(ch, b)
        out_copy(ch, b).start()
        prev = ch - 1
        nxt = ch + _NBUF - 1

        @pl.when(jnp.logical_and(prev >= 0, nxt < _NCHUNK))
        def _():
            out_copy(prev, (b - 1) % _NBUF).wait()
            in_copy(nxt, (b - 1) % _NBUF).start()

    for b in range(_NBUF):
        in_copy(b, b).start()

    _NMAIN = (_NCHUNK // _NBUF) * _NBUF

    @pl.loop(0, _NMAIN, step=_NBUF)
    def _(g):
        for b in range(_NBUF):
            step(g + b, b)

    for ch in range(_NMAIN, _NCHUNK):
        step(ch, ch % _NBUF)

    for ch in range(_NCHUNK - _NBUF, _NCHUNK):
        out_copy(ch, ch % _NBUF).wait()


def kernel(input_tensor, mask_tensor):
    B, T, F = input_tensor.shape
    x = input_tensor.reshape(_N, _F)
    m = mask_tensor.reshape(_N)
    mesh = plsc.VectorSubcoreMesh(core_axis_name="c", subcore_axis_name="s")
    out = pl.kernel(
        _sc_body,
        out_type=jax.ShapeDtypeStruct((_N, _F), jnp.float32),
        mesh=mesh,
        scratch_types=[
            pltpu.VMEM((_NBUF, _CH, _F), jnp.float32),
            pltpu.VMEM((_RPW,), jnp.float32),
            pltpu.SemaphoreType.DMA((_NBUF,)),
            pltpu.SemaphoreType.DMA((_NBUF,)),
        ],
    )(x, m)
    return out.reshape(B, T, F)
